# baseline (speedup 1.0000x reference)
import functools

import jax
import jax.numpy as jnp
from jax.experimental import pallas as pl

_NA = 3
_NC = 10


def _head_body(x_ref, w_ref, b_ref, o_ref):
    x = x_ref[0]
    for a in range(_NA):
        y = jax.lax.dot_general(
            x, w_ref[a],
            dimension_numbers=(((0,), (0,)), ((), ())),
            preferred_element_type=jnp.float32,
        )
        o_ref[0, a] = y + b_ref[a][None, :]


@functools.partial(jax.jit, static_argnames=("tile",))
def _head(x, W, b, tile):
    B, C, H, Wd = x.shape
    hw = H * Wd
    xf = x.reshape(B, C, hw)
    Wa = jnp.transpose(W.reshape(_NA, _NC, C), (0, 2, 1))
    ba = b.reshape(_NA, _NC)
    nt = hw // tile
    out = pl.pallas_call(
        _head_body,
        grid=(B, nt),
        in_specs=[
            pl.BlockSpec((1, C, tile), lambda bi, ti: (bi, 0, ti)),
            pl.BlockSpec((_NA, C, _NC), lambda bi, ti: (0, 0, 0)),
            pl.BlockSpec((_NA, _NC), lambda bi, ti: (0, 0)),
        ],
        out_specs=pl.BlockSpec((1, _NA, tile, _NC), lambda bi, ti: (bi, 0, ti, 0)),
        out_shape=jax.ShapeDtypeStruct((B, _NA, hw, _NC), jnp.float32),
    )(xf, Wa, ba)
    return out.reshape(B, _NA, H, Wd, _NC)


def kernel(p3, p4, p5, W1, b1, W2, b2, W3, b3):
    o3 = _head(p3, W1, b1, 1280)
    o4 = _head(p4, W2, b2, 1600)
    o5 = _head(p5, W3, b3, 400)
    return (o3, o4, o5)
